# baseline (speedup 1.0000x reference)
import functools

import jax
import jax.numpy as jnp
from jax import lax
from jax.experimental import pallas as pl
from jax.experimental.pallas import tpu as pltpu
from jax.experimental.pallas import tpu_sc as plsc

NUM_VQ = 4
D = 768
L = 16
NC, NS = 2, 16
NW = NC * NS
CHT = 64
CHC = 16
DL = D // L


def _sc_kernel(P, V, ids_hbm, mask_hbm, text_hbm, code_hbm, out_hbm,
               ids_v, mask_v, tpos_v, tidx_v, cpos_v, cidx_v,
               tpos_s0, tpos_s1, cpos_s0, cpos_s1, cb0, cb1, sg0, sg1, ss0, ss1):
    tpos_sb = (tpos_s0, tpos_s1)
    cpos_sb = (cpos_s0, cpos_s1)
    PW = P // NW
    GROUPS = PW // L
    CPL = PW + CHC
    DUMMY = P
    wid = lax.axis_index("s") * NC + lax.axis_index("c")
    base_p = wid * PW
    iota = lax.iota(jnp.int32, L)
    bufs = ((cb0, sg0, ss0), (cb1, sg1, ss1))

    for i in range(NUM_VQ):
        pltpu.sync_copy(ids_hbm.at[pl.ds(i * P + base_p, PW)],
                        ids_v.at[pl.ds(i * PW, PW)])
    pltpu.sync_copy(mask_hbm.at[pl.ds(base_p, PW)], mask_v)

    zeros = jnp.zeros((L,), jnp.int32)
    dummy = jnp.full((L,), DUMMY, jnp.int32)

    def fill_t(i, _):
        tpos_v[pl.ds(i * L, L)] = dummy
        tidx_v[pl.ds(i * L, L)] = zeros
        return 0
    lax.fori_loop(0, (PW + CHT) // L, fill_t, 0)

    def fill_c(i, _):
        cpos_v[pl.ds(i * L, L)] = dummy
        for j in range(NUM_VQ):
            cidx_v[pl.ds(j * CPL + i * L, L)] = zeros
        return 0
    lax.fori_loop(0, CPL // L, fill_c, 0)

    def compact(g, carry):
        nt, nc = carry
        p0 = g * L
        m16 = mask_v[pl.ds(p0, L)]
        tmask = m16 > 0
        cmask = m16 == 0
        gpos = base_p + p0 + iota
        tid16 = ids_v[pl.ds(p0, L)]
        plsc.store_compressed(tidx_v.at[pl.ds(nt, L)], tid16, mask=tmask)
        plsc.store_compressed(tpos_v.at[pl.ds(nt, L)], gpos, mask=tmask)
        plsc.store_compressed(cpos_v.at[pl.ds(nc, L)], gpos, mask=cmask)
        for i in range(NUM_VQ):
            civ = ids_v[pl.ds(i * PW + p0, L)] + i * V
            plsc.store_compressed(cidx_v.at[pl.ds(i * CPL + nc, L)], civ,
                                  mask=cmask)
        tcnt = jnp.sum(m16)
        return nt + tcnt, nc + (L - tcnt)

    nt, nc = lax.fori_loop(0, GROUPS, compact, (jnp.int32(0), jnp.int32(0)))
    nct = (nt + CHT - 1) // CHT
    ncc = (nc + CHC - 1) // CHC

    def t_gather(k, buf, sem):
        return pltpu.make_async_copy(
            text_hbm.at[tidx_v.at[pl.ds(k * CHT, CHT)]], buf, sem)

    def t_scatter(par, buf, sem):
        return pltpu.make_async_copy(buf, out_hbm.at[tpos_sb[par]], sem)

    @pl.when(nct > 0)
    def _():
        t_gather(0, cb0, sg0).start()

    def tbody(k, _):
        for par in range(2):
            kk = 2 * k + par
            buf, sg, ss = bufs[par]
            obuf, osg, oss = bufs[1 - par]

            @pl.when(kk < nct)
            def _():
                t_gather(kk, buf, sg).wait()
                off = kk * CHT
                for j in range(CHT // L):
                    tpos_sb[par][pl.ds(j * L, L)] = (
                        tpos_v[pl.ds(off + j * L, L)])

                @pl.when(kk >= 1)
                def _():
                    t_scatter(1 - par, obuf, oss).wait()

                @pl.when(kk + 1 < nct)
                def _():
                    t_gather(kk + 1, obuf, osg).start()
                t_scatter(par, buf, ss).start()
        return 0
    lax.fori_loop(0, (nct + 1) // 2, tbody, 0)
    for par in range(2):
        buf, sg, ss = bufs[par]

        @pl.when((nct >= 1) & (((nct - 1) & 1) == par))
        def _():
            t_scatter(par, buf, ss).wait()

    def c_gathers(k, buf, sem):
        off = k * CHC
        return [pltpu.make_async_copy(
            code_hbm.at[cidx_v.at[pl.ds(i * CPL + off, CHC)]],
            buf.at[pl.ds(i * CHC, CHC)], sem) for i in range(NUM_VQ)]

    def c_scatter(par, buf, sem):
        return pltpu.make_async_copy(
            buf.at[pl.ds(0, CHC)], out_hbm.at[cpos_sb[par]], sem)

    @pl.when(ncc > 0)
    def _():
        for d in c_gathers(0, cb0, sg0):
            d.start()

    def cbody(k, _):
        for par in range(2):
            kk = 2 * k + par
            buf, sg, ss = bufs[par]
            obuf, osg, oss = bufs[1 - par]

            @pl.when(kk < ncc)
            def _():
                for d in c_gathers(kk, buf, sg):
                    d.wait()
                cpos_sb[par][pl.ds(0, L)] = cpos_v[pl.ds(kk * CHC, L)]

                @pl.when(kk >= 1)
                def _():
                    c_scatter(1 - par, obuf, oss).wait()

                @pl.when(kk + 1 < ncc)
                def _():
                    for d in c_gathers(kk + 1, obuf, osg):
                        d.start()

                def sum4(p, _):
                    for dd in range(DL):
                        s = pl.ds(dd * L, L)
                        buf[p, s] = (buf[p, s] + buf[CHC + p, s]
                                     + buf[2 * CHC + p, s]
                                     + buf[3 * CHC + p, s])
                    return 0
                lax.fori_loop(0, CHC, sum4, 0)
                c_scatter(par, buf, ss).start()
        return 0
    lax.fori_loop(0, (ncc + 1) // 2, cbody, 0)
    for par in range(2):
        buf, sg, ss = bufs[par]

        @pl.when((ncc >= 1) & (((ncc - 1) & 1) == par))
        def _():
            c_scatter(par, buf, ss).wait()


def kernel(input_ids, text_mask, emb_text_table, emb_code_tables):
    B, S, _ = input_ids.shape
    P = B * S
    PW = P // NW
    V = emb_code_tables.shape[1]
    ids_t = jnp.transpose(input_ids.reshape(P, NUM_VQ)).reshape(NUM_VQ * P)
    ids_t = ids_t.astype(jnp.int32)
    mask_flat = text_mask.reshape(P).astype(jnp.int32)
    code_flat = emb_code_tables.reshape(NUM_VQ * V, D)

    mesh = plsc.VectorSubcoreMesh(core_axis_name="c", subcore_axis_name="s",
                                  num_cores=NC, num_subcores=NS)
    run = pl.kernel(
        functools.partial(_sc_kernel, P, V),
        out_type=jax.ShapeDtypeStruct((P + 8, D), jnp.float32),
        mesh=mesh,
        compiler_params=pltpu.CompilerParams(needs_layout_passes=False),
        scratch_types=[
            pltpu.VMEM((NUM_VQ * PW,), jnp.int32),
            pltpu.VMEM((PW,), jnp.int32),
            pltpu.VMEM((PW + CHT,), jnp.int32),
            pltpu.VMEM((PW + CHT,), jnp.int32),
            pltpu.VMEM((PW + CHC,), jnp.int32),
            pltpu.VMEM((NUM_VQ * (PW + CHC),), jnp.int32),
            pltpu.VMEM((CHT,), jnp.int32),
            pltpu.VMEM((CHT,), jnp.int32),
            pltpu.VMEM((L,), jnp.int32),
            pltpu.VMEM((L,), jnp.int32),
            pltpu.VMEM((NUM_VQ * CHC, D), jnp.float32),
            pltpu.VMEM((NUM_VQ * CHC, D), jnp.float32),
            pltpu.SemaphoreType.DMA,
            pltpu.SemaphoreType.DMA,
            pltpu.SemaphoreType.DMA,
            pltpu.SemaphoreType.DMA,
        ],
    )
    out = run(ids_t, mask_flat, emb_text_table, code_flat)
    return out[:P].reshape(B, S, D)


# speedup vs baseline: 1.8565x; 1.8565x over previous
import functools

import jax
import jax.numpy as jnp
from jax import lax
from jax.experimental import pallas as pl
from jax.experimental.pallas import tpu as pltpu
from jax.experimental.pallas import tpu_sc as plsc

NUM_VQ = 4
D = 768
L = 16
NC, NS = 2, 16
NW = NC * NS
CHT = 64
CHC = 16
DL = D // L


def _sc_kernel(P, V, ids_hbm, mask_hbm, text_hbm, code_hbm, out_hbm,
               ids_v, mask_v, tpos_v, tidx_v, cpos_v, cidx_v,
               tpos_s0, tpos_s1, cpos_s0, cpos_s1, cidx_s0, cidx_s1,
               cb0, cb1, sg0, sg1, ss0, ss1):
    tpos_sb = (tpos_s0, tpos_s1)
    cpos_sb = (cpos_s0, cpos_s1)
    cidx_sb = (cidx_s0, cidx_s1)
    PW = P // NW
    GROUPS = PW // L
    CPL = PW + CHC
    wid = lax.axis_index("s") * NC + lax.axis_index("c")
    base_p = wid * PW
    iota = lax.iota(jnp.int32, L)
    bufs = ((cb0, sg0, ss0), (cb1, sg1, ss1))

    for i in range(NUM_VQ):
        pltpu.sync_copy(ids_hbm.at[pl.ds(i * P + base_p, PW)],
                        ids_v.at[pl.ds(i * PW, PW)])
    pltpu.sync_copy(mask_hbm.at[pl.ds(base_p, PW)], mask_v)

    def compact(g, carry):
        nt, nc = carry
        p0 = g * L
        m16 = mask_v[pl.ds(p0, L)]
        tmask = m16 > 0
        cmask = m16 == 0
        gpos = base_p + p0 + iota
        tid16 = ids_v[pl.ds(p0, L)]
        plsc.store_compressed(tidx_v.at[pl.ds(nt, L)], tid16, mask=tmask)
        plsc.store_compressed(tpos_v.at[pl.ds(nt, L)], gpos, mask=tmask)
        plsc.store_compressed(cpos_v.at[pl.ds(nc, L)], gpos, mask=cmask)
        for i in range(NUM_VQ):
            civ = ids_v[pl.ds(i * PW + p0, L)] + i * V
            plsc.store_compressed(cidx_v.at[pl.ds(i * CPL + nc, L)], civ,
                                  mask=cmask)
        tcnt = jnp.sum(m16)
        return nt + tcnt, nc + (L - tcnt)

    nt, nc = lax.fori_loop(0, GROUPS, compact, (jnp.int32(0), jnp.int32(0)))
    nct = (nt + CHT - 1) // CHT
    ncc = (nc + CHC - 1) // CHC

    @pl.when(nt > 0)
    def _():
        tiv = jnp.broadcast_to(tidx_v[pl.ds(0, L)][0], (L,))
        tpv = jnp.broadcast_to(tpos_v[pl.ds(0, L)][0], (L,))
        for j in range(CHT // L):
            tidx_v[pl.ds(nt + j * L, L)] = tiv
            tpos_v[pl.ds(nt + j * L, L)] = tpv

    @pl.when(nc > 0)
    def _():
        cpv = jnp.broadcast_to(cpos_v[pl.ds(0, L)][0], (L,))
        cpos_v[pl.ds(nc, L)] = cpv
        for i in range(NUM_VQ):
            civ = jnp.broadcast_to(cidx_v[pl.ds(i * CPL, L)][0], (L,))
            cidx_v[pl.ds(i * CPL + nc, L)] = civ

    def t_gather(k, buf, sem):
        return pltpu.make_async_copy(
            text_hbm.at[tidx_v.at[pl.ds(k * CHT, CHT)]], buf, sem)

    def t_scatter(par, buf, sem):
        return pltpu.make_async_copy(buf, out_hbm.at[tpos_sb[par]], sem)

    @pl.when(nct > 0)
    def _():
        t_gather(0, cb0, sg0).start()

    def tbody(k, _):
        for par in range(2):
            kk = 2 * k + par
            buf, sg, ss = bufs[par]
            obuf, osg, oss = bufs[1 - par]

            @pl.when(kk < nct)
            def _():
                t_gather(kk, buf, sg).wait()
                off = kk * CHT
                for j in range(CHT // L):
                    tpos_sb[par][pl.ds(j * L, L)] = (
                        tpos_v[pl.ds(off + j * L, L)])

                @pl.when(kk >= 1)
                def _():
                    t_scatter(1 - par, obuf, oss).wait()

                @pl.when(kk + 1 < nct)
                def _():
                    t_gather(kk + 1, obuf, osg).start()
                t_scatter(par, buf, ss).start()
        return 0
    lax.fori_loop(0, (nct + 1) // 2, tbody, 0)
    for par in range(2):
        buf, sg, ss = bufs[par]

        @pl.when((nct >= 1) & (((nct - 1) & 1) == par))
        def _():
            t_scatter(par, buf, ss).wait()

    def c_prep(k, par):
        off = k * CHC
        for i in range(NUM_VQ):
            cidx_sb[par][pl.ds(i * CHC, L)] = cidx_v[pl.ds(i * CPL + off, L)]

    def c_gather(par, buf, sem):
        return pltpu.make_async_copy(code_hbm.at[cidx_sb[par]], buf, sem)

    def c_scatter(par, buf, sem):
        return pltpu.make_async_copy(
            buf.at[pl.ds(0, CHC)], out_hbm.at[cpos_sb[par]], sem)

    @pl.when(ncc > 0)
    def _():
        c_prep(0, 0)
        c_gather(0, cb0, sg0).start()

    def cbody(k, _):
        for par in range(2):
            kk = 2 * k + par
            buf, sg, ss = bufs[par]
            obuf, osg, oss = bufs[1 - par]

            @pl.when(kk < ncc)
            def _():
                c_gather(par, buf, sg).wait()
                cpos_sb[par][pl.ds(0, L)] = cpos_v[pl.ds(kk * CHC, L)]

                @pl.when(kk >= 1)
                def _():
                    c_scatter(1 - par, obuf, oss).wait()

                @pl.when(kk + 1 < ncc)
                def _():
                    c_prep(kk + 1, 1 - par)
                    c_gather(1 - par, obuf, osg).start()

                def sum4(p, _):
                    for dd in range(DL):
                        s = pl.ds(dd * L, L)
                        buf[p, s] = (buf[p, s] + buf[CHC + p, s]
                                     + buf[2 * CHC + p, s]
                                     + buf[3 * CHC + p, s])
                    return 0
                lax.fori_loop(0, CHC, sum4, 0)
                c_scatter(par, buf, ss).start()
        return 0
    lax.fori_loop(0, (ncc + 1) // 2, cbody, 0)
    for par in range(2):
        buf, sg, ss = bufs[par]

        @pl.when((ncc >= 1) & (((ncc - 1) & 1) == par))
        def _():
            c_scatter(par, buf, ss).wait()


def kernel(input_ids, text_mask, emb_text_table, emb_code_tables):
    B, S, _ = input_ids.shape
    P = B * S
    PW = P // NW
    V = emb_code_tables.shape[1]
    ids_t = jnp.transpose(input_ids.reshape(P, NUM_VQ)).reshape(NUM_VQ * P)
    ids_t = ids_t.astype(jnp.int32)
    mask_flat = text_mask.reshape(P).astype(jnp.int32)
    code_flat = emb_code_tables.reshape(NUM_VQ * V, D)

    mesh = plsc.VectorSubcoreMesh(core_axis_name="c", subcore_axis_name="s",
                                  num_cores=NC, num_subcores=NS)
    run = pl.kernel(
        functools.partial(_sc_kernel, P, V),
        out_type=jax.ShapeDtypeStruct((P, D), jnp.float32),
        mesh=mesh,
        compiler_params=pltpu.CompilerParams(needs_layout_passes=False),
        scratch_types=[
            pltpu.VMEM((NUM_VQ * PW,), jnp.int32),
            pltpu.VMEM((PW,), jnp.int32),
            pltpu.VMEM((PW + CHT,), jnp.int32),
            pltpu.VMEM((PW + CHT,), jnp.int32),
            pltpu.VMEM((PW + CHC,), jnp.int32),
            pltpu.VMEM((NUM_VQ * (PW + CHC),), jnp.int32),
            pltpu.VMEM((CHT,), jnp.int32),
            pltpu.VMEM((CHT,), jnp.int32),
            pltpu.VMEM((L,), jnp.int32),
            pltpu.VMEM((L,), jnp.int32),
            pltpu.VMEM((NUM_VQ * CHC,), jnp.int32),
            pltpu.VMEM((NUM_VQ * CHC,), jnp.int32),
            pltpu.VMEM((NUM_VQ * CHC, D), jnp.float32),
            pltpu.VMEM((NUM_VQ * CHC, D), jnp.float32),
            pltpu.SemaphoreType.DMA,
            pltpu.SemaphoreType.DMA,
            pltpu.SemaphoreType.DMA,
            pltpu.SemaphoreType.DMA,
        ],
    )
    out = run(ids_t, mask_flat, emb_text_table, code_flat)
    return out.reshape(B, S, D)


# speedup vs baseline: 1.8717x; 1.0082x over previous
import functools

import jax
import jax.numpy as jnp
from jax import lax
from jax.experimental import pallas as pl
from jax.experimental.pallas import tpu as pltpu
from jax.experimental.pallas import tpu_sc as plsc

NUM_VQ = 4
D = 768
L = 16
NC, NS = 2, 16
NW = NC * NS
CHT = 64
CHC = 16
DL = D // L


def _sc_kernel(P, V, ids_hbm, mask_hbm, text_hbm, code_hbm, out_hbm,
               ids_v, mask_v, tpos_v, tidx_v, cpos_v, cidx_v,
               tpos_s0, tpos_s1, cpos_s0, cpos_s1, cidx_s0, cidx_s1,
               cb0, cb1, sg0, sg1, ss0, ss1):
    tpos_sb = (tpos_s0, tpos_s1)
    cpos_sb = (cpos_s0, cpos_s1)
    cidx_sb = (cidx_s0, cidx_s1)
    PW = P // NW
    GROUPS = PW // L
    CPL = PW + CHC
    wid = lax.axis_index("s") * NC + lax.axis_index("c")
    base_p = wid * PW
    iota = lax.iota(jnp.int32, L)
    bufs = ((cb0, sg0, ss0), (cb1, sg1, ss1))

    stage = [pltpu.make_async_copy(ids_hbm.at[pl.ds(i * P + base_p, PW)],
                                   ids_v.at[pl.ds(i * PW, PW)], sg0)
             for i in range(NUM_VQ)]
    stage.append(pltpu.make_async_copy(mask_hbm.at[pl.ds(base_p, PW)],
                                       mask_v, sg0))
    for d in stage:
        d.start()
    for d in stage:
        d.wait()

    def compact(g, carry):
        nt, nc = carry
        p0 = g * L
        m16 = mask_v[pl.ds(p0, L)]
        tmask = m16 > 0
        cmask = m16 == 0
        gpos = base_p + p0 + iota
        tid16 = ids_v[pl.ds(p0, L)]
        plsc.store_compressed(tidx_v.at[pl.ds(nt, L)], tid16, mask=tmask)
        plsc.store_compressed(tpos_v.at[pl.ds(nt, L)], gpos, mask=tmask)
        plsc.store_compressed(cpos_v.at[pl.ds(nc, L)], gpos, mask=cmask)
        for i in range(NUM_VQ):
            civ = ids_v[pl.ds(i * PW + p0, L)] + i * V
            plsc.store_compressed(cidx_v.at[pl.ds(i * CPL + nc, L)], civ,
                                  mask=cmask)
        tcnt = jnp.sum(m16)
        return nt + tcnt, nc + (L - tcnt)

    nt, nc = lax.fori_loop(0, GROUPS, compact, (jnp.int32(0), jnp.int32(0)))
    nct = (nt + CHT - 1) // CHT
    ncc = (nc + CHC - 1) // CHC

    @pl.when(nt > 0)
    def _():
        tiv = jnp.broadcast_to(tidx_v[pl.ds(0, L)][0], (L,))
        tpv = jnp.broadcast_to(tpos_v[pl.ds(0, L)][0], (L,))
        for j in range(CHT // L):
            tidx_v[pl.ds(nt + j * L, L)] = tiv
            tpos_v[pl.ds(nt + j * L, L)] = tpv

    @pl.when(nc > 0)
    def _():
        cpv = jnp.broadcast_to(cpos_v[pl.ds(0, L)][0], (L,))
        cpos_v[pl.ds(nc, L)] = cpv
        for i in range(NUM_VQ):
            civ = jnp.broadcast_to(cidx_v[pl.ds(i * CPL, L)][0], (L,))
            cidx_v[pl.ds(i * CPL + nc, L)] = civ

    def t_gather(k, buf, sem):
        return pltpu.make_async_copy(
            text_hbm.at[tidx_v.at[pl.ds(k * CHT, CHT)]], buf, sem)

    def t_scatter(par, buf, sem):
        return pltpu.make_async_copy(buf, out_hbm.at[tpos_sb[par]], sem)

    @pl.when(nct > 0)
    def _():
        t_gather(0, cb0, sg0).start()

    def tbody(k, _):
        for par in range(2):
            kk = 2 * k + par
            buf, sg, ss = bufs[par]
            obuf, osg, oss = bufs[1 - par]

            @pl.when(kk < nct)
            def _():
                t_gather(kk, buf, sg).wait()
                off = kk * CHT
                for j in range(CHT // L):
                    tpos_sb[par][pl.ds(j * L, L)] = (
                        tpos_v[pl.ds(off + j * L, L)])

                @pl.when(kk >= 1)
                def _():
                    t_scatter(1 - par, obuf, oss).wait()

                @pl.when(kk + 1 < nct)
                def _():
                    t_gather(kk + 1, obuf, osg).start()
                t_scatter(par, buf, ss).start()
        return 0
    lax.fori_loop(0, (nct + 1) // 2, tbody, 0)
    for par in range(2):
        buf, sg, ss = bufs[par]

        @pl.when((nct >= 1) & (((nct - 1) & 1) == par))
        def _():
            t_scatter(par, buf, ss).wait()

    def c_prep(k, par):
        off = k * CHC
        for i in range(NUM_VQ):
            cidx_sb[par][pl.ds(i * CHC, L)] = cidx_v[pl.ds(i * CPL + off, L)]

    def c_gather(par, buf, sem):
        return pltpu.make_async_copy(code_hbm.at[cidx_sb[par]], buf, sem)

    def c_scatter(par, buf, sem):
        return pltpu.make_async_copy(
            buf.at[pl.ds(0, CHC)], out_hbm.at[cpos_sb[par]], sem)

    @pl.when(ncc > 0)
    def _():
        c_prep(0, 0)
        c_gather(0, cb0, sg0).start()

    def cbody(k, _):
        for par in range(2):
            kk = 2 * k + par
            buf, sg, ss = bufs[par]
            obuf, osg, oss = bufs[1 - par]

            @pl.when(kk < ncc)
            def _():
                c_gather(par, buf, sg).wait()
                cpos_sb[par][pl.ds(0, L)] = cpos_v[pl.ds(kk * CHC, L)]

                @pl.when(kk >= 1)
                def _():
                    c_scatter(1 - par, obuf, oss).wait()

                @pl.when(kk + 1 < ncc)
                def _():
                    c_prep(kk + 1, 1 - par)
                    c_gather(1 - par, obuf, osg).start()

                def sum4(p, _):
                    for dd in range(DL):
                        s = pl.ds(dd * L, L)
                        buf[p, s] = (buf[p, s] + buf[CHC + p, s]
                                     + buf[2 * CHC + p, s]
                                     + buf[3 * CHC + p, s])
                    return 0
                lax.fori_loop(0, CHC, sum4, 0)
                c_scatter(par, buf, ss).start()
        return 0
    lax.fori_loop(0, (ncc + 1) // 2, cbody, 0)
    for par in range(2):
        buf, sg, ss = bufs[par]

        @pl.when((ncc >= 1) & (((ncc - 1) & 1) == par))
        def _():
            c_scatter(par, buf, ss).wait()


def kernel(input_ids, text_mask, emb_text_table, emb_code_tables):
    B, S, _ = input_ids.shape
    P = B * S
    PW = P // NW
    V = emb_code_tables.shape[1]
    ids_t = jnp.transpose(input_ids.reshape(P, NUM_VQ)).reshape(NUM_VQ * P)
    ids_t = ids_t.astype(jnp.int32)
    mask_flat = text_mask.reshape(P).astype(jnp.int32)
    code_flat = emb_code_tables.reshape(NUM_VQ * V, D)

    mesh = plsc.VectorSubcoreMesh(core_axis_name="c", subcore_axis_name="s",
                                  num_cores=NC, num_subcores=NS)
    run = pl.kernel(
        functools.partial(_sc_kernel, P, V),
        out_type=jax.ShapeDtypeStruct((P, D), jnp.float32),
        mesh=mesh,
        compiler_params=pltpu.CompilerParams(needs_layout_passes=False),
        scratch_types=[
            pltpu.VMEM((NUM_VQ * PW,), jnp.int32),
            pltpu.VMEM((PW,), jnp.int32),
            pltpu.VMEM((PW + CHT,), jnp.int32),
            pltpu.VMEM((PW + CHT,), jnp.int32),
            pltpu.VMEM((PW + CHC,), jnp.int32),
            pltpu.VMEM((NUM_VQ * (PW + CHC),), jnp.int32),
            pltpu.VMEM((CHT,), jnp.int32),
            pltpu.VMEM((CHT,), jnp.int32),
            pltpu.VMEM((L,), jnp.int32),
            pltpu.VMEM((L,), jnp.int32),
            pltpu.VMEM((NUM_VQ * CHC,), jnp.int32),
            pltpu.VMEM((NUM_VQ * CHC,), jnp.int32),
            pltpu.VMEM((NUM_VQ * CHC, D), jnp.float32),
            pltpu.VMEM((NUM_VQ * CHC, D), jnp.float32),
            pltpu.SemaphoreType.DMA,
            pltpu.SemaphoreType.DMA,
            pltpu.SemaphoreType.DMA,
            pltpu.SemaphoreType.DMA,
        ],
    )
    out = run(ids_t, mask_flat, emb_text_table, code_flat)
    return out.reshape(B, S, D)
